# baseline (speedup 1.0000x reference)
import functools

import jax
import jax.numpy as jnp
from jax import lax
from jax.experimental import pallas as pl
from jax.experimental.pallas import tpu as pltpu
from jax.experimental.pallas import tpu_sc as plsc

N = 10000
E = 320000
D = 128
NG = 64

NC, NS = 2, 16
NW = NC * NS
CH = 128
CHUNKS = 80
K0, K1 = 34, 126
E_PAD = NW * CH * CHUNKS
ACC = 10240
ZR = ACC // NS
GPAD = 10016
RB = 1000

_mesh = plsc.VectorSubcoreMesh(core_axis_name="c", subcore_axis_name="s")


@functools.partial(
    pl.kernel,
    out_type=jax.ShapeDtypeStruct((NC, ACC, D), jnp.float32),
    mesh=_mesh,
    scratch_types=[
        pltpu.VMEM((CHUNKS, CH), jnp.int32),
        pltpu.VMEM((CH, D), jnp.float32),
        pltpu.VMEM_SHARED((ACC, D), jnp.float32),
    ],
)
def _deg_kernel(dst_hbm, ones_hbm, zeros_hbm, out_hbm, idx_d, rows, acc):
    c = lax.axis_index("c")
    s = lax.axis_index("s")
    w = s * NC + c
    pltpu.sync_copy(ones_hbm, rows)
    pltpu.sync_copy(dst_hbm.at[w], idx_d)
    pltpu.sync_copy(zeros_hbm, acc.at[pl.ds(s * ZR, ZR)])
    plsc.subcore_barrier()

    @pl.loop(0, CHUNKS)
    def _chunk(j):
        pltpu.sync_copy(rows, acc.at[idx_d.at[j]], add=True)

    plsc.subcore_barrier()
    pltpu.sync_copy(acc.at[pl.ds(s * ZR, ZR)], out_hbm.at[c, pl.ds(s * ZR, ZR)])


@functools.partial(
    pl.kernel,
    out_type=jax.ShapeDtypeStruct((NC, ACC, D), jnp.float32),
    mesh=_mesh,
    scratch_types=[
        pltpu.VMEM((CH,), jnp.int32),
        pltpu.VMEM((CH,), jnp.int32),
        pltpu.VMEM((CH, D), jnp.float32),
        pltpu.VMEM_SHARED((ACC, D), jnp.float32),
        pltpu.SemaphoreType.DMA,
    ],
)
def _agg_kernel(g_hbm, src_hbm, dst_hbm, zeros_hbm, out_hbm,
                idx_s, idx_d, rows, acc, sem):
    c = lax.axis_index("c")
    s = lax.axis_index("s")
    w = s * NC + c
    base = w * CHUNKS * CH
    pltpu.sync_copy(zeros_hbm, acc.at[pl.ds(s * ZR, ZR)])
    plsc.subcore_barrier()

    @pl.loop(0, CHUNKS)
    def _chunk(j):
        b = base + j * CH
        pltpu.sync_copy(src_hbm.at[pl.ds(b, CH)], idx_s)
        pltpu.sync_copy(dst_hbm.at[pl.ds(b, CH)], idx_d)
        pltpu.async_copy(g_hbm.at[idx_s], rows, sem).wait()
        pltpu.sync_copy(rows, acc.at[idx_d], add=True)

    plsc.subcore_barrier()
    pltpu.sync_copy(acc.at[pl.ds(s * ZR, ZR)], out_hbm.at[c, pl.ds(s * ZR, ZR)])


def _prep_body(x_ref, w_ref, d0_ref, d1_ref, dinv_ref, g_ref):
    deg = d0_ref[...] + d1_ref[...] + 1.0
    dinv = lax.rsqrt(deg)
    dinv_ref[...] = dinv
    g_ref[...] = dinv * jnp.dot(x_ref[...], w_ref[...],
                                preferred_element_type=jnp.float32)


def _mid_body(p0_ref, p1_ref, g_ref, dinv_ref, b_ref, w_ref, gout_ref):
    dinv = dinv_ref[...]
    h = jnp.maximum(
        dinv * (p0_ref[...] + p1_ref[...] + g_ref[...]) + b_ref[...], 0.0)
    gout_ref[...] = dinv * jnp.dot(h, w_ref[...],
                                   preferred_element_type=jnp.float32)


def _final_body(p0_ref, p1_ref, g_ref, dinv_ref, b_ref, batch_ref,
                wfc_ref, bfc_ref, out_ref, sums, counts):
    i = pl.program_id(0)

    @pl.when(i == 0)
    def _init():
        sums[...] = jnp.zeros_like(sums)
        counts[...] = jnp.zeros_like(counts)

    h = jnp.maximum(
        dinv_ref[...] * (p0_ref[...] + p1_ref[...] + g_ref[...])
        + b_ref[...], 0.0)
    bb = batch_ref[0, 0, :]
    gid = lax.broadcasted_iota(jnp.int32, (NG, RB), 0)
    onehot = jnp.where(bb[None, :] == gid, 1.0, 0.0)
    sums[...] += jnp.dot(onehot, h, preferred_element_type=jnp.float32)
    counts[...] += jnp.dot(onehot, jnp.ones((RB, D), jnp.float32),
                           preferred_element_type=jnp.float32)

    @pl.when(i == pl.num_programs(0) - 1)
    def _emit():
        pooled = sums[...] / jnp.maximum(counts[...], 1.0)
        out_ref[...] = jnp.dot(pooled, wfc_ref[...],
                               preferred_element_type=jnp.float32) + bfc_ref[...]


_row = pl.BlockSpec((RB, D), lambda i: (i, 0))
_full = pl.BlockSpec((D, D), lambda i: (0, 0))
_bias = pl.BlockSpec((1, D), lambda i: (0, 0))

_prep = pl.pallas_call(
    _prep_body,
    grid=(N // RB,),
    in_specs=[_row, _full, _row, _row],
    out_specs=[_row, _row],
    out_shape=[jax.ShapeDtypeStruct((N, D), jnp.float32),
               jax.ShapeDtypeStruct((N, D), jnp.float32)],
)

_mid = pl.pallas_call(
    _mid_body,
    grid=(N // RB,),
    in_specs=[_row, _row, _row, _row, _bias, _full],
    out_specs=_row,
    out_shape=jax.ShapeDtypeStruct((N, D), jnp.float32),
)

_final = pl.pallas_call(
    _final_body,
    grid=(N // RB,),
    in_specs=[_row, _row, _row, _row, _bias,
              pl.BlockSpec((1, 1, RB), lambda i: (i, 0, 0)),
              _full, _bias],
    out_specs=pl.BlockSpec((NG, D), lambda i: (0, 0)),
    out_shape=jax.ShapeDtypeStruct((NG, D), jnp.float32),
    scratch_shapes=[pltpu.VMEM((NG, D), jnp.float32),
                    pltpu.VMEM((NG, D), jnp.float32)],
)


def _pad_rows(g):
    return jnp.pad(g, ((0, GPAD - N), (0, 0)))


def kernel(x, edge_index, edge_attr, batch, W1, b1, W2, b2, W3, b3, Wfc, bfc):
    del edge_attr
    src = edge_index[0].astype(jnp.int32)
    dst = edge_index[1].astype(jnp.int32)
    tail = jnp.full((E_PAD - E,), N, jnp.int32)
    src_p = jnp.concatenate([src, tail])
    dst_p = jnp.concatenate([dst, tail])
    dst_3d = dst_p.reshape(NW, CHUNKS, CH)

    ones_rows = jnp.ones((CH, D), jnp.float32)
    zeros_rows = jnp.zeros((ZR, D), jnp.float32)

    dpart = _deg_kernel(dst_3d, ones_rows, zeros_rows)
    dinvb, g = _prep(x, W1, dpart[0], dpart[1])

    for W_next, b_prev in ((W2, b1), (W3, b2)):
        p = _agg_kernel(_pad_rows(g), src_p, dst_p, zeros_rows)
        g = _mid(p[0], p[1], g, dinvb, b_prev.reshape(1, D), W_next)

    p = _agg_kernel(_pad_rows(g), src_p, dst_p, zeros_rows)
    wfc_p = jnp.pad(Wfc, ((0, 0), (0, D - Wfc.shape[1])))
    bfc_p = jnp.pad(bfc, (0, D - bfc.shape[0])).reshape(1, D)
    batch3 = batch.astype(jnp.int32).reshape(N // RB, 1, RB)
    out = _final(p[0], p[1], g, dinvb, b3.reshape(1, D), batch3, wfc_p, bfc_p)
    return out[:, :bfc.shape[0]]


# speedup vs baseline: 1.4304x; 1.4304x over previous
import functools

import jax
import jax.numpy as jnp
from jax import lax
from jax.experimental import pallas as pl
from jax.experimental.pallas import tpu as pltpu
from jax.experimental.pallas import tpu_sc as plsc

N = 10000
E = 320000
D = 128
NG = 64

NC, NS = 2, 16
NW = NC * NS
CH = 128
CHUNKS = 79
E_PAD = NW * CH * CHUNKS
ACC = 10240
ZR = ACC // NS
GPAD = 10016
RB = 1000

_mesh = plsc.VectorSubcoreMesh(core_axis_name="c", subcore_axis_name="s")


@functools.partial(
    pl.kernel,
    out_type=jax.ShapeDtypeStruct((NC, ACC, D), jnp.float32),
    mesh=_mesh,
    scratch_types=[
        pltpu.VMEM((CH,), jnp.int32),
        pltpu.VMEM((CH, D), jnp.float32),
        pltpu.VMEM_SHARED((ACC, D), jnp.float32),
    ],
)
def _deg_kernel(dst_hbm, ones_hbm, zeros_hbm, out_hbm, idx_d, rows, acc):
    c = lax.axis_index("c")
    s = lax.axis_index("s")
    w = s * NC + c
    pltpu.sync_copy(ones_hbm, rows)
    pltpu.sync_copy(zeros_hbm, acc.at[pl.ds(s * ZR, ZR)])
    plsc.subcore_barrier()

    @pl.loop(0, CHUNKS)
    def _chunk(j):
        pltpu.sync_copy(dst_hbm.at[pl.ds((w * CHUNKS + j) * CH, CH)], idx_d)
        pltpu.sync_copy(rows, acc.at[idx_d], add=True)

    plsc.subcore_barrier()
    pltpu.sync_copy(acc.at[pl.ds(s * ZR, ZR)], out_hbm.at[c, pl.ds(s * ZR, ZR)])


@functools.partial(
    pl.kernel,
    out_type=jax.ShapeDtypeStruct((NC, ACC, D), jnp.float32),
    mesh=_mesh,
    scratch_types=[
        pltpu.VMEM((CH,), jnp.int32),
        pltpu.VMEM((CH,), jnp.int32),
        pltpu.VMEM((CH, D), jnp.float32),
        pltpu.VMEM_SHARED((ACC, D), jnp.float32),
        pltpu.SemaphoreType.DMA,
    ],
)
def _agg_kernel(g_hbm, src_hbm, dst_hbm, zeros_hbm, out_hbm,
                idx_s, idx_d, rows, acc, sem):
    c = lax.axis_index("c")
    s = lax.axis_index("s")
    w = s * NC + c
    base = w * CHUNKS * CH
    pltpu.sync_copy(zeros_hbm, acc.at[pl.ds(s * ZR, ZR)])
    plsc.subcore_barrier()

    @pl.loop(0, CHUNKS)
    def _chunk(j):
        b = base + j * CH
        pltpu.sync_copy(src_hbm.at[pl.ds(b, CH)], idx_s)
        pltpu.sync_copy(dst_hbm.at[pl.ds(b, CH)], idx_d)
        pltpu.async_copy(g_hbm.at[idx_s], rows, sem).wait()
        pltpu.sync_copy(rows, acc.at[idx_d], add=True)

    plsc.subcore_barrier()
    pltpu.sync_copy(acc.at[pl.ds(s * ZR, ZR)], out_hbm.at[c, pl.ds(s * ZR, ZR)])


def _prep_body(x_ref, w_ref, d0_ref, d1_ref, dinv_ref, g_ref):
    deg = d0_ref[...] + d1_ref[...] + 1.0
    dinv = lax.rsqrt(deg)
    dinv_ref[...] = dinv
    g_ref[...] = dinv * jnp.dot(x_ref[...], w_ref[...],
                                preferred_element_type=jnp.float32)


def _mid_body(p0_ref, p1_ref, g_ref, dinv_ref, b_ref, w_ref, gout_ref):
    dinv = dinv_ref[...]
    h = jnp.maximum(
        dinv * (p0_ref[...] + p1_ref[...] + g_ref[...]) + b_ref[...], 0.0)
    gout_ref[...] = dinv * jnp.dot(h, w_ref[...],
                                   preferred_element_type=jnp.float32)


def _final_body(p0_ref, p1_ref, g_ref, dinv_ref, b_ref, batch_ref,
                wfc_ref, bfc_ref, out_ref, sums, counts):
    i = pl.program_id(0)

    @pl.when(i == 0)
    def _init():
        sums[...] = jnp.zeros_like(sums)
        counts[...] = jnp.zeros_like(counts)

    h = jnp.maximum(
        dinv_ref[...] * (p0_ref[...] + p1_ref[...] + g_ref[...])
        + b_ref[...], 0.0)
    bb = batch_ref[0, 0, :]
    gid = lax.broadcasted_iota(jnp.int32, (NG, RB), 0)
    onehot = jnp.where(bb[None, :] == gid, 1.0, 0.0)
    sums[...] += jnp.dot(onehot, h, preferred_element_type=jnp.float32)
    counts[...] += jnp.dot(onehot, jnp.ones((RB, D), jnp.float32),
                           preferred_element_type=jnp.float32)

    @pl.when(i == pl.num_programs(0) - 1)
    def _emit():
        pooled = sums[...] / jnp.maximum(counts[...], 1.0)
        out_ref[...] = jnp.dot(pooled, wfc_ref[...],
                               preferred_element_type=jnp.float32) + bfc_ref[...]


_row = pl.BlockSpec((RB, D), lambda i: (i, 0))
_full = pl.BlockSpec((D, D), lambda i: (0, 0))
_bias = pl.BlockSpec((1, D), lambda i: (0, 0))

_prep = pl.pallas_call(
    _prep_body,
    grid=(N // RB,),
    in_specs=[_row, _full, _row, _row],
    out_specs=[_row, _row],
    out_shape=[jax.ShapeDtypeStruct((N, D), jnp.float32),
               jax.ShapeDtypeStruct((N, D), jnp.float32)],
)

_mid = pl.pallas_call(
    _mid_body,
    grid=(N // RB,),
    in_specs=[_row, _row, _row, _row, _bias, _full],
    out_specs=_row,
    out_shape=jax.ShapeDtypeStruct((N, D), jnp.float32),
)

_final = pl.pallas_call(
    _final_body,
    grid=(N // RB,),
    in_specs=[_row, _row, _row, _row, _bias,
              pl.BlockSpec((1, 1, RB), lambda i: (i, 0, 0)),
              _full, _bias],
    out_specs=pl.BlockSpec((NG, D), lambda i: (0, 0)),
    out_shape=jax.ShapeDtypeStruct((NG, D), jnp.float32),
    scratch_shapes=[pltpu.VMEM((NG, D), jnp.float32),
                    pltpu.VMEM((NG, D), jnp.float32)],
)


def _pad_rows(g):
    return jnp.pad(g, ((0, GPAD - N), (0, 0)))


def kernel(x, edge_index, edge_attr, batch, W1, b1, W2, b2, W3, b3, Wfc, bfc):
    del edge_attr
    src = edge_index[0].astype(jnp.int32)
    dst = edge_index[1].astype(jnp.int32)
    tail = jnp.full((E_PAD - E,), N, jnp.int32)
    src_p = jnp.concatenate([src, tail])
    dst_p = jnp.concatenate([dst, tail])

    ones_rows = jnp.ones((CH, D), jnp.float32)
    zeros_rows = jnp.zeros((ZR, D), jnp.float32)

    dpart = _deg_kernel(dst_p, ones_rows, zeros_rows)
    dinvb, g = _prep(x, W1, dpart[0], dpart[1])

    for W_next, b_prev in ((W2, b1), (W3, b2)):
        p = _agg_kernel(_pad_rows(g), src_p, dst_p, zeros_rows)
        g = _mid(p[0], p[1], g, dinvb, b_prev.reshape(1, D), W_next)

    p = _agg_kernel(_pad_rows(g), src_p, dst_p, zeros_rows)
    wfc_p = jnp.pad(Wfc, ((0, 0), (0, D - Wfc.shape[1])))
    bfc_p = jnp.pad(bfc, (0, D - bfc.shape[0])).reshape(1, D)
    batch3 = batch.astype(jnp.int32).reshape(N // RB, 1, RB)
    out = _final(p[0], p[1], g, dinvb, b3.reshape(1, D), batch3, wfc_p, bfc_p)
    return out[:, :bfc.shape[0]]
